# baseline (speedup 1.0000x reference)
import functools
import jax
import jax.numpy as jnp
from jax import lax
from jax.experimental import pallas as pl
from jax.experimental.pallas import tpu as pltpu
from jax.experimental.pallas import tpu_sc as plsc

BLK = 128
TI = 2048
_NC, _NS = 2, 16
_NW = _NC * _NS


def _sc_dispatch(x, disp_tok, cap):
    t, h = x.shape
    per_w = cap // _NW
    ch = per_w // 2
    nch = per_w // ch
    mesh = plsc.VectorSubcoreMesh(core_axis_name="c", subcore_axis_name="s")

    @functools.partial(
        pl.kernel,
        out_type=jax.ShapeDtypeStruct((cap, h), jnp.float32),
        mesh=mesh,
        scratch_types=[
            pltpu.VMEM((ch,), jnp.int32),
            pltpu.VMEM((ch, h), jnp.float32),
            pltpu.SemaphoreType.DMA,
        ],
    )
    def k(x_hbm, tok_hbm, xg_hbm, idx_v, rows_v, sem):
        wid = lax.axis_index("s") * _NC + lax.axis_index("c")
        for c in range(nch):
            base = wid * per_w + c * ch
            pltpu.sync_copy(tok_hbm.at[pl.ds(base, ch)], idx_v)
            pltpu.async_copy(x_hbm.at[idx_v], rows_v, sem).wait()
            pltpu.sync_copy(rows_v, xg_hbm.at[pl.ds(base, ch)])

    return k(x, disp_tok)


def _sc_combine(y, p1, p2):
    t = p1.shape[0]
    h = y.shape[1]
    per_w = t // _NW
    ncol = h // 16
    mesh = plsc.VectorSubcoreMesh(core_axis_name="c", subcore_axis_name="s")

    @functools.partial(
        pl.kernel,
        out_type=jax.ShapeDtypeStruct((t, h), jnp.float32),
        mesh=mesh,
        scratch_types=[
            pltpu.VMEM((per_w,), jnp.int32),
            pltpu.VMEM((per_w,), jnp.int32),
            pltpu.VMEM((per_w, h), jnp.float32),
            pltpu.VMEM((per_w, h), jnp.float32),
            pltpu.SemaphoreType.DMA,
        ],
    )
    def k(y_hbm, p1_hbm, p2_hbm, out_hbm, i1_v, i2_v, r1_v, r2_v, sem):
        wid = lax.axis_index("s") * _NC + lax.axis_index("c")
        base = wid * per_w
        pltpu.sync_copy(p1_hbm.at[pl.ds(base, per_w)], i1_v)
        pltpu.sync_copy(p2_hbm.at[pl.ds(base, per_w)], i2_v)
        c1 = pltpu.async_copy(y_hbm.at[i1_v], r1_v, sem)
        c2 = pltpu.async_copy(y_hbm.at[i2_v], r2_v, sem)
        c1.wait()
        c2.wait()

        def body(tok, carry):
            for c in range(ncol):
                sl = pl.ds(c * 16, 16)
                r1_v[tok, sl] = r1_v[tok, sl] + r2_v[tok, sl]
            return carry

        lax.fori_loop(0, per_w, body, 0)
        pltpu.sync_copy(r1_v, out_hbm.at[pl.ds(base, per_w)])

    return k(y, p1, p2)


def _router_kernel(x_ref, gw_ref, logits_ref, meta_ref):
    x = x_ref[...]
    gw = gw_ref[...]
    l = jnp.dot(x, gw, preferred_element_type=jnp.float32)
    logits_ref[...] = l
    lane = jax.lax.broadcasted_iota(jnp.int32, l.shape, 1)
    valid = lane < 8
    neg = jnp.float32(-jnp.inf)
    lm = jnp.where(valid, l, neg)
    m1 = jnp.max(lm, axis=1, keepdims=True)
    i1 = jnp.min(jnp.where(lm == m1, lane, 128), axis=1, keepdims=True)
    s = jnp.sum(jnp.where(valid, jnp.exp(lm - m1), 0.0), axis=1, keepdims=True)
    lm2 = jnp.where(lane == i1, neg, lm)
    m2 = jnp.max(lm2, axis=1, keepdims=True)
    i2 = jnp.min(jnp.where(lm2 == m2, lane, 128), axis=1, keepdims=True)
    w1 = 1.0 / s
    w2 = jnp.exp(m2 - m1) / s
    meta = (jnp.where(lane == 0, i1.astype(jnp.float32), 0.0)
            + jnp.where(lane == 1, i2.astype(jnp.float32), 0.0)
            + jnp.where(lane == 2, w1, 0.0)
            + jnp.where(lane == 3, w2, 0.0))
    meta_ref[...] = meta


def _ffn_kernel(be_ref, xg_ref, win_ref, wv_ref, wout_ref, wt_ref, y_ref,
                winb_ref, wvb_ref, woutb_ref):
    i = pl.program_id(0)
    bb = pl.program_id(1)
    ni = pl.num_programs(0)
    prev = be_ref[jnp.maximum(bb - 1, 0)]
    changed = jnp.logical_or(bb == 0, be_ref[bb] != prev)

    @pl.when(changed)
    def _():
        winb_ref[...] = win_ref[0].astype(jnp.bfloat16)
        wvb_ref[...] = wv_ref[0].astype(jnp.bfloat16)
        woutb_ref[...] = wout_ref[0].astype(jnp.bfloat16)

    xb = xg_ref[...].astype(jnp.bfloat16)
    up = jnp.dot(xb, winb_ref[...], preferred_element_type=jnp.float32)
    v = jnp.dot(xb, wvb_ref[...], preferred_element_type=jnp.float32)
    hg = (jax.nn.gelu(up) * v).astype(jnp.bfloat16)
    part = jnp.dot(hg, woutb_ref[...], preferred_element_type=jnp.float32)

    rows = pl.ds(bb * BLK, BLK)
    if ni == 1:
        y_ref[rows, :] = part * wt_ref[0]
    else:
        @pl.when(i == 0)
        def _():
            y_ref[rows, :] = part

        @pl.when(jnp.logical_and(i > 0, i < ni - 1))
        def _():
            y_ref[rows, :] += part

        @pl.when(i == ni - 1)
        def _():
            y_ref[rows, :] = (y_ref[rows, :] + part) * wt_ref[0]


def kernel(hidden, gate_w, w_in, w_v, w_out):
    b, s, h = hidden.shape
    e = gate_w.shape[1]
    ii = w_in.shape[2]
    t = b * s
    x = hidden.reshape(t, h)

    gw_pad = jnp.zeros((h, 128), jnp.float32).at[:, :e].set(gate_w)
    rblk = 256
    logits_pad, meta = pl.pallas_call(
        _router_kernel,
        grid=(t // rblk,),
        in_specs=[
            pl.BlockSpec((rblk, h), lambda i: (i, 0)),
            pl.BlockSpec((h, 128), lambda i: (0, 0)),
        ],
        out_specs=[
            pl.BlockSpec((rblk, 128), lambda i: (i, 0)),
            pl.BlockSpec((rblk, 128), lambda i: (i, 0)),
        ],
        out_shape=[
            jax.ShapeDtypeStruct((t, 128), jnp.float32),
            jax.ShapeDtypeStruct((t, 128), jnp.float32),
        ],
    )(x, gw_pad)
    router_logits = logits_pad[:, :e]
    i1 = meta[:, 0].astype(jnp.int32)
    i2 = meta[:, 1].astype(jnp.int32)
    w1 = meta[:, 2]
    w2 = meta[:, 3]

    cap = 2 * t + e * BLK
    nb = cap // BLK
    ids = jnp.concatenate([i1, i2])
    toks = jnp.concatenate([jnp.arange(t, dtype=jnp.int32)] * 2)
    wts = jnp.concatenate([w1, w2])
    onehot = (ids[:, None] == jnp.arange(e, dtype=jnp.int32)[None, :])
    csum = jnp.cumsum(onehot.astype(jnp.int32), axis=0)
    rank = jnp.sum(csum * onehot, axis=1) - 1
    counts = csum[-1]
    padded = ((counts + BLK - 1) // BLK) * BLK
    offs = jnp.concatenate([jnp.zeros((1,), jnp.int32),
                            jnp.cumsum(padded)[:-1].astype(jnp.int32)])
    pos = offs[ids] + rank
    disp_tok = jnp.zeros((cap,), jnp.int32).at[pos].set(toks)
    disp_wt = jnp.zeros((cap,), jnp.float32).at[pos].set(wts)
    blk_offs = offs // BLK
    block_expert = jnp.sum(
        jnp.arange(nb, dtype=jnp.int32)[:, None] >= blk_offs[None, :],
        axis=1).astype(jnp.int32) - 1
    block_expert = jnp.clip(block_expert, 0, e - 1)
    p1, p2 = pos[:t], pos[t:]

    xg = _sc_dispatch(x, disp_tok, cap)

    ni = ii // TI
    wtb = disp_wt.reshape(nb, BLK, 1)
    grid_spec = pltpu.PrefetchScalarGridSpec(
        num_scalar_prefetch=1,
        grid=(ni, nb),
        in_specs=[
            pl.BlockSpec((BLK, h), lambda i, bb, be: (bb, 0)),
            pl.BlockSpec((1, h, TI), lambda i, bb, be: (be[bb], 0, i)),
            pl.BlockSpec((1, h, TI), lambda i, bb, be: (be[bb], 0, i)),
            pl.BlockSpec((1, TI, h), lambda i, bb, be: (be[bb], i, 0)),
            pl.BlockSpec((1, BLK, 1), lambda i, bb, be: (bb, 0, 0)),
        ],
        out_specs=pl.BlockSpec((cap, h), lambda i, bb, be: (0, 0)),
        scratch_shapes=[
            pltpu.VMEM((h, TI), jnp.bfloat16),
            pltpu.VMEM((h, TI), jnp.bfloat16),
            pltpu.VMEM((TI, h), jnp.bfloat16),
        ],
    )
    y = pl.pallas_call(
        _ffn_kernel,
        grid_spec=grid_spec,
        out_shape=jax.ShapeDtypeStruct((cap, h), jnp.float32),
        compiler_params=pltpu.CompilerParams(
            dimension_semantics=("arbitrary", "arbitrary"),
            vmem_limit_bytes=120 * 1024 * 1024),
    )(block_expert, xg, w_in, w_v, w_out, wtb)

    out = _sc_combine(y, p1, p2)
    return out.reshape(b, s, h), router_logits


# speedup vs baseline: 1.0354x; 1.0354x over previous
import functools
import jax
import jax.numpy as jnp
from jax import lax
from jax.experimental import pallas as pl
from jax.experimental.pallas import tpu as pltpu
from jax.experimental.pallas import tpu_sc as plsc

BLK = 128
TI = 2048
_NC, _NS = 2, 16
_NW = _NC * _NS


def _sc_dispatch(x, pos, cap):
    t, h = x.shape
    na = 2 * t
    per_w = na // _NW
    mesh = plsc.VectorSubcoreMesh(core_axis_name="c", subcore_axis_name="s")

    @functools.partial(
        pl.kernel,
        out_type=jax.ShapeDtypeStruct((cap, h), jnp.float32),
        mesh=mesh,
        scratch_types=[
            pltpu.VMEM((per_w,), jnp.int32),
            pltpu.VMEM((per_w, h), jnp.float32),
            pltpu.SemaphoreType.DMA,
        ],
    )
    def k(x_hbm, pos_hbm, xg_hbm, pos_v, rows_v, sem):
        wid = lax.axis_index("s") * _NC + lax.axis_index("c")
        a_base = wid * per_w
        tok_base = lax.rem(a_base, t)
        pltpu.sync_copy(pos_hbm.at[pl.ds(a_base, per_w)], pos_v)
        pltpu.sync_copy(x_hbm.at[pl.ds(tok_base, per_w)], rows_v)
        pltpu.async_copy(rows_v, xg_hbm.at[pos_v], sem).wait()

    return k(x, pos)


def _sc_combine(y, p1, p2, w1, w2):
    t = p1.shape[0]
    h = y.shape[1]
    per_w = t // _NW
    ncol = h // 16
    mesh = plsc.VectorSubcoreMesh(core_axis_name="c", subcore_axis_name="s")

    @functools.partial(
        pl.kernel,
        out_type=jax.ShapeDtypeStruct((t, h), jnp.float32),
        mesh=mesh,
        scratch_types=[
            pltpu.VMEM((per_w,), jnp.int32),
            pltpu.VMEM((per_w,), jnp.int32),
            pltpu.VMEM((per_w,), jnp.float32),
            pltpu.VMEM((per_w,), jnp.float32),
            pltpu.VMEM((per_w, h), jnp.float32),
            pltpu.VMEM((per_w, h), jnp.float32),
            pltpu.SemaphoreType.DMA,
        ],
        compiler_params=pltpu.CompilerParams(needs_layout_passes=False),
    )
    def k(y_hbm, p1_hbm, p2_hbm, w1_hbm, w2_hbm, out_hbm,
          i1_v, i2_v, a1_v, a2_v, r1_v, r2_v, sem):
        wid = lax.axis_index("s") * _NC + lax.axis_index("c")
        base = wid * per_w
        pltpu.sync_copy(p1_hbm.at[pl.ds(base, per_w)], i1_v)
        pltpu.sync_copy(p2_hbm.at[pl.ds(base, per_w)], i2_v)
        pltpu.sync_copy(w1_hbm.at[pl.ds(base, per_w)], a1_v)
        pltpu.sync_copy(w2_hbm.at[pl.ds(base, per_w)], a2_v)
        c1 = pltpu.async_copy(y_hbm.at[i1_v], r1_v, sem)
        c2 = pltpu.async_copy(y_hbm.at[i2_v], r2_v, sem)
        c1.wait()
        c2.wait()

        def body(tok, carry):
            splat = jnp.full((16,), tok, jnp.int32)
            s1 = plsc.load_gather(a1_v, [splat])
            s2 = plsc.load_gather(a2_v, [splat])
            for c in range(ncol):
                sl = pl.ds(c * 16, 16)
                r1_v[tok, sl] = s1 * r1_v[tok, sl] + s2 * r2_v[tok, sl]
            return carry

        lax.fori_loop(0, per_w, body, 0)
        pltpu.sync_copy(r1_v, out_hbm.at[pl.ds(base, per_w)])

    return k(y, p1, p2, w1, w2)


def _router_kernel(x_ref, gw_ref, logits_ref, meta_ref):
    x = x_ref[...]
    gw = gw_ref[...]
    l = jnp.dot(x, gw, preferred_element_type=jnp.float32)
    logits_ref[...] = l
    lane = jax.lax.broadcasted_iota(jnp.int32, l.shape, 1)
    valid = lane < 8
    neg = jnp.float32(-jnp.inf)
    lm = jnp.where(valid, l, neg)
    m1 = jnp.max(lm, axis=1, keepdims=True)
    i1 = jnp.min(jnp.where(lm == m1, lane, 128), axis=1, keepdims=True)
    s = jnp.sum(jnp.where(valid, jnp.exp(lm - m1), 0.0), axis=1, keepdims=True)
    lm2 = jnp.where(lane == i1, neg, lm)
    m2 = jnp.max(lm2, axis=1, keepdims=True)
    i2 = jnp.min(jnp.where(lm2 == m2, lane, 128), axis=1, keepdims=True)
    w1 = 1.0 / s
    w2 = jnp.exp(m2 - m1) / s
    meta = (jnp.where(lane == 0, i1.astype(jnp.float32), 0.0)
            + jnp.where(lane == 1, i2.astype(jnp.float32), 0.0)
            + jnp.where(lane == 2, w1, 0.0)
            + jnp.where(lane == 3, w2, 0.0))
    meta_ref[...] = meta


def _ffn_kernel(be_ref, xg_ref, win_ref, wv_ref, wout_ref, y_ref,
                winb_ref, wvb_ref, woutb_ref):
    i = pl.program_id(0)
    bb = pl.program_id(1)
    ni = pl.num_programs(0)
    prev = be_ref[jnp.maximum(bb - 1, 0)]
    changed = jnp.logical_or(bb == 0, be_ref[bb] != prev)

    @pl.when(changed)
    def _():
        winb_ref[...] = win_ref[0].astype(jnp.bfloat16)
        wvb_ref[...] = wv_ref[0].astype(jnp.bfloat16)
        woutb_ref[...] = wout_ref[0].astype(jnp.bfloat16)

    xb = xg_ref[...].astype(jnp.bfloat16)
    up = jnp.dot(xb, winb_ref[...], preferred_element_type=jnp.float32)
    v = jnp.dot(xb, wvb_ref[...], preferred_element_type=jnp.float32)
    hg = (jax.nn.gelu(up) * v).astype(jnp.bfloat16)
    part = jnp.dot(hg, woutb_ref[...], preferred_element_type=jnp.float32)

    rows = pl.ds(bb * BLK, BLK)
    if ni == 1:
        y_ref[rows, :] = part
    else:
        @pl.when(i == 0)
        def _():
            y_ref[rows, :] = part

        @pl.when(i > 0)
        def _():
            y_ref[rows, :] += part


def kernel(hidden, gate_w, w_in, w_v, w_out):
    b, s, h = hidden.shape
    e = gate_w.shape[1]
    ii = w_in.shape[2]
    t = b * s
    x = hidden.reshape(t, h)

    gw_pad = jnp.zeros((h, 128), jnp.float32).at[:, :e].set(gate_w)
    rblk = 256
    logits_pad, meta = pl.pallas_call(
        _router_kernel,
        grid=(t // rblk,),
        in_specs=[
            pl.BlockSpec((rblk, h), lambda i: (i, 0)),
            pl.BlockSpec((h, 128), lambda i: (0, 0)),
        ],
        out_specs=[
            pl.BlockSpec((rblk, 128), lambda i: (i, 0)),
            pl.BlockSpec((rblk, 128), lambda i: (i, 0)),
        ],
        out_shape=[
            jax.ShapeDtypeStruct((t, 128), jnp.float32),
            jax.ShapeDtypeStruct((t, 128), jnp.float32),
        ],
    )(x, gw_pad)
    router_logits = logits_pad[:, :e]
    i1 = meta[:, 0].astype(jnp.int32)
    i2 = meta[:, 1].astype(jnp.int32)
    w1 = meta[:, 2]
    w2 = meta[:, 3]

    cap = 2 * t + e * BLK
    nb = cap // BLK
    ids = jnp.concatenate([i1, i2])
    onehot = (ids[:, None] == jnp.arange(e, dtype=jnp.int32)[None, :])
    csum = jnp.cumsum(onehot.astype(jnp.int32), axis=0)
    rank = jnp.sum(csum * onehot, axis=1) - 1
    counts = csum[-1]
    padded = ((counts + BLK - 1) // BLK) * BLK
    offs = jnp.concatenate([jnp.zeros((1,), jnp.int32),
                            jnp.cumsum(padded)[:-1].astype(jnp.int32)])
    pos = offs[ids] + rank
    blk_offs = offs // BLK
    block_expert = jnp.sum(
        jnp.arange(nb, dtype=jnp.int32)[:, None] >= blk_offs[None, :],
        axis=1).astype(jnp.int32) - 1
    block_expert = jnp.clip(block_expert, 0, e - 1)
    p1, p2 = pos[:t], pos[t:]

    xg = _sc_dispatch(x, pos, cap)

    ni = ii // TI
    grid_spec = pltpu.PrefetchScalarGridSpec(
        num_scalar_prefetch=1,
        grid=(ni, nb),
        in_specs=[
            pl.BlockSpec((BLK, h), lambda i, bb, be: (bb, 0)),
            pl.BlockSpec((1, h, TI), lambda i, bb, be: (be[bb], 0, i)),
            pl.BlockSpec((1, h, TI), lambda i, bb, be: (be[bb], 0, i)),
            pl.BlockSpec((1, TI, h), lambda i, bb, be: (be[bb], i, 0)),
        ],
        out_specs=pl.BlockSpec((cap, h), lambda i, bb, be: (0, 0)),
        scratch_shapes=[
            pltpu.VMEM((h, TI), jnp.bfloat16),
            pltpu.VMEM((h, TI), jnp.bfloat16),
            pltpu.VMEM((TI, h), jnp.bfloat16),
        ],
    )
    y = pl.pallas_call(
        _ffn_kernel,
        grid_spec=grid_spec,
        out_shape=jax.ShapeDtypeStruct((cap, h), jnp.float32),
        compiler_params=pltpu.CompilerParams(
            dimension_semantics=("arbitrary", "arbitrary"),
            vmem_limit_bytes=120 * 1024 * 1024),
    )(block_expert, xg, w_in, w_v, w_out)

    out = _sc_combine(y, p1, p2, w1, w2)
    return out.reshape(b, s, h), router_logits


# speedup vs baseline: 1.0743x; 1.0376x over previous
import functools
import jax
import jax.numpy as jnp
from jax import lax
from jax.experimental import pallas as pl
from jax.experimental.pallas import tpu as pltpu
from jax.experimental.pallas import tpu_sc as plsc

BLK = 128
TI = 2048
_NC, _NS = 2, 16
_NW = _NC * _NS


def _sc_dispatch(x, pos, cap):
    t, h = x.shape
    na = 2 * t
    per_w = na // _NW
    mesh = plsc.VectorSubcoreMesh(core_axis_name="c", subcore_axis_name="s")

    @functools.partial(
        pl.kernel,
        out_type=jax.ShapeDtypeStruct((cap, h), jnp.float32),
        mesh=mesh,
        scratch_types=[
            pltpu.VMEM((per_w,), jnp.int32),
            pltpu.VMEM((per_w, h), jnp.float32),
            pltpu.SemaphoreType.DMA,
        ],
    )
    def k(x_hbm, pos_hbm, xg_hbm, pos_v, rows_v, sem):
        wid = lax.axis_index("s") * _NC + lax.axis_index("c")
        a_base = wid * per_w
        tok_base = lax.rem(a_base, t)
        pltpu.sync_copy(pos_hbm.at[pl.ds(a_base, per_w)], pos_v)
        pltpu.sync_copy(x_hbm.at[pl.ds(tok_base, per_w)], rows_v)
        pltpu.async_copy(rows_v, xg_hbm.at[pos_v], sem).wait()

    return k(x, pos)


def _sc_combine(y, p1, p2, w1, w2):
    t = p1.shape[0]
    h = y.shape[1]
    per_w = t // _NW
    ncol = h // 16
    mesh = plsc.VectorSubcoreMesh(core_axis_name="c", subcore_axis_name="s")

    @functools.partial(
        pl.kernel,
        out_type=jax.ShapeDtypeStruct((t, h), jnp.float32),
        mesh=mesh,
        scratch_types=[
            pltpu.VMEM((per_w,), jnp.int32),
            pltpu.VMEM((per_w,), jnp.int32),
            pltpu.VMEM((per_w,), jnp.float32),
            pltpu.VMEM((per_w,), jnp.float32),
            pltpu.VMEM((per_w, h), jnp.float32),
            pltpu.VMEM((per_w, h), jnp.float32),
            pltpu.SemaphoreType.DMA,
        ],
        compiler_params=pltpu.CompilerParams(needs_layout_passes=False),
    )
    def k(y_hbm, p1_hbm, p2_hbm, w1_hbm, w2_hbm, out_hbm,
          i1_v, i2_v, a1_v, a2_v, r1_v, r2_v, sem):
        wid = lax.axis_index("s") * _NC + lax.axis_index("c")
        base = wid * per_w
        pltpu.sync_copy(p1_hbm.at[pl.ds(base, per_w)], i1_v)
        pltpu.sync_copy(p2_hbm.at[pl.ds(base, per_w)], i2_v)
        pltpu.sync_copy(w1_hbm.at[pl.ds(base, per_w)], a1_v)
        pltpu.sync_copy(w2_hbm.at[pl.ds(base, per_w)], a2_v)
        c1 = pltpu.async_copy(y_hbm.at[i1_v], r1_v, sem)
        c2 = pltpu.async_copy(y_hbm.at[i2_v], r2_v, sem)
        c1.wait()
        c2.wait()

        def body(tok, carry):
            splat = jnp.full((16,), tok, jnp.int32)
            s1 = plsc.load_gather(a1_v, [splat])
            s2 = plsc.load_gather(a2_v, [splat])
            for c in range(ncol):
                sl = pl.ds(c * 16, 16)
                r1_v[tok, sl] = s1 * r1_v[tok, sl] + s2 * r2_v[tok, sl]
            return carry

        lax.fori_loop(0, per_w, body, 0)
        pltpu.sync_copy(r1_v, out_hbm.at[pl.ds(base, per_w)])

    return k(y, p1, p2, w1, w2)


def _router_kernel(x_ref, gw_ref, logits_ref, meta_ref):
    x = x_ref[...]
    gw = gw_ref[...]
    l = jnp.dot(x, gw, preferred_element_type=jnp.float32)
    logits_ref[...] = l
    lane = jax.lax.broadcasted_iota(jnp.int32, l.shape, 1)
    valid = lane < 8
    neg = jnp.float32(-jnp.inf)
    lm = jnp.where(valid, l, neg)
    m1 = jnp.max(lm, axis=1, keepdims=True)
    i1 = jnp.min(jnp.where(lm == m1, lane, 128), axis=1, keepdims=True)
    s = jnp.sum(jnp.where(valid, jnp.exp(lm - m1), 0.0), axis=1, keepdims=True)
    lm2 = jnp.where(lane == i1, neg, lm)
    m2 = jnp.max(lm2, axis=1, keepdims=True)
    i2 = jnp.min(jnp.where(lm2 == m2, lane, 128), axis=1, keepdims=True)
    w1 = 1.0 / s
    w2 = jnp.exp(m2 - m1) / s
    meta = (jnp.where(lane == 0, i1.astype(jnp.float32), 0.0)
            + jnp.where(lane == 1, i2.astype(jnp.float32), 0.0)
            + jnp.where(lane == 2, w1, 0.0)
            + jnp.where(lane == 3, w2, 0.0))
    meta_ref[...] = meta


def _ffn_kernel(be_ref, nbu_ref, xg_ref, win_ref, wv_ref, wout_ref, y_ref,
                winb_ref, wvb_ref, woutb_ref):
    i = pl.program_id(0)
    bb = pl.program_id(1)
    ni = pl.num_programs(0)
    valid = bb < nbu_ref[0]
    prev = be_ref[jnp.maximum(bb - 1, 0)]
    changed = jnp.logical_or(bb == 0, be_ref[bb] != prev)

    @pl.when(jnp.logical_and(changed, valid))
    def _():
        winb_ref[...] = win_ref[0].astype(jnp.bfloat16)
        wvb_ref[...] = wv_ref[0].astype(jnp.bfloat16)
        woutb_ref[...] = wout_ref[0].astype(jnp.bfloat16)

    @pl.when(valid)
    def _():
        xb = xg_ref[...].astype(jnp.bfloat16)
        up = jnp.dot(xb, winb_ref[...], preferred_element_type=jnp.float32)
        v = jnp.dot(xb, wvb_ref[...], preferred_element_type=jnp.float32)
        hg = (jax.nn.gelu(up) * v).astype(jnp.bfloat16)
        part = jnp.dot(hg, woutb_ref[...], preferred_element_type=jnp.float32)

        rows = pl.ds(bb * BLK, BLK)
        if ni == 1:
            y_ref[rows, :] = part
        else:
            @pl.when(i == 0)
            def _():
                y_ref[rows, :] = part

            @pl.when(i > 0)
            def _():
                y_ref[rows, :] += part


def kernel(hidden, gate_w, w_in, w_v, w_out):
    b, s, h = hidden.shape
    e = gate_w.shape[1]
    ii = w_in.shape[2]
    t = b * s
    x = hidden.reshape(t, h)

    gw_pad = jnp.zeros((h, 128), jnp.float32).at[:, :e].set(gate_w)
    rblk = 256
    logits_pad, meta = pl.pallas_call(
        _router_kernel,
        grid=(t // rblk,),
        in_specs=[
            pl.BlockSpec((rblk, h), lambda i: (i, 0)),
            pl.BlockSpec((h, 128), lambda i: (0, 0)),
        ],
        out_specs=[
            pl.BlockSpec((rblk, 128), lambda i: (i, 0)),
            pl.BlockSpec((rblk, 128), lambda i: (i, 0)),
        ],
        out_shape=[
            jax.ShapeDtypeStruct((t, 128), jnp.float32),
            jax.ShapeDtypeStruct((t, 128), jnp.float32),
        ],
    )(x, gw_pad)
    router_logits = logits_pad[:, :e]
    i1 = meta[:, 0].astype(jnp.int32)
    i2 = meta[:, 1].astype(jnp.int32)
    w1 = meta[:, 2]
    w2 = meta[:, 3]

    cap = 2 * t + e * BLK
    nb = cap // BLK
    ids = jnp.concatenate([i1, i2])
    onehot = (ids[:, None] == jnp.arange(e, dtype=jnp.int32)[None, :])
    csum = jnp.cumsum(onehot.astype(jnp.int32), axis=0)
    rank = jnp.sum(csum * onehot, axis=1) - 1
    counts = csum[-1]
    padded = ((counts + BLK - 1) // BLK) * BLK
    offs = jnp.concatenate([jnp.zeros((1,), jnp.int32),
                            jnp.cumsum(padded)[:-1].astype(jnp.int32)])
    pos = offs[ids] + rank
    blk_offs = offs // BLK
    block_expert = jnp.sum(
        jnp.arange(nb, dtype=jnp.int32)[:, None] >= blk_offs[None, :],
        axis=1).astype(jnp.int32) - 1
    block_expert = jnp.clip(block_expert, 0, e - 1)
    nb_used = jnp.sum(padded, dtype=jnp.int32) // BLK
    nb_used = nb_used.reshape(1)
    p1, p2 = pos[:t], pos[t:]

    xg = _sc_dispatch(x, pos, cap)

    ni = ii // TI
    grid_spec = pltpu.PrefetchScalarGridSpec(
        num_scalar_prefetch=2,
        grid=(ni, nb),
        in_specs=[
            pl.BlockSpec((BLK, h), lambda i, bb, be, nbu: (bb, 0)),
            pl.BlockSpec((1, h, TI), lambda i, bb, be, nbu: (be[bb], 0, i)),
            pl.BlockSpec((1, h, TI), lambda i, bb, be, nbu: (be[bb], 0, i)),
            pl.BlockSpec((1, TI, h), lambda i, bb, be, nbu: (be[bb], i, 0)),
        ],
        out_specs=pl.BlockSpec((cap, h), lambda i, bb, be, nbu: (0, 0)),
        scratch_shapes=[
            pltpu.VMEM((h, TI), jnp.bfloat16),
            pltpu.VMEM((h, TI), jnp.bfloat16),
            pltpu.VMEM((TI, h), jnp.bfloat16),
        ],
    )
    y = pl.pallas_call(
        _ffn_kernel,
        grid_spec=grid_spec,
        out_shape=jax.ShapeDtypeStruct((cap, h), jnp.float32),
        compiler_params=pltpu.CompilerParams(
            dimension_semantics=("arbitrary", "arbitrary"),
            vmem_limit_bytes=120 * 1024 * 1024),
    )(block_expert, nb_used, xg, w_in, w_v, w_out)

    out = _sc_combine(y, p1, p2, w1, w2)
    return out.reshape(b, s, h), router_logits


# speedup vs baseline: 1.1343x; 1.0559x over previous
import functools
import jax
import jax.numpy as jnp
from jax import lax
from jax.experimental import pallas as pl
from jax.experimental.pallas import tpu as pltpu
from jax.experimental.pallas import tpu_sc as plsc

BLK = 128
TI = 2048
_NC, _NS = 2, 16
_NW = _NC * _NS


def _sc_dispatch(x, pos, cap):
    t, h = x.shape
    na = 2 * t
    per_w = na // _NW
    mesh = plsc.VectorSubcoreMesh(core_axis_name="c", subcore_axis_name="s")

    @functools.partial(
        pl.kernel,
        out_type=jax.ShapeDtypeStruct((cap, h), jnp.float32),
        mesh=mesh,
        scratch_types=[
            pltpu.VMEM((per_w,), jnp.int32),
            pltpu.VMEM((per_w, h), jnp.float32),
            pltpu.SemaphoreType.DMA,
        ],
    )
    def k(x_hbm, pos_hbm, xg_hbm, pos_v, rows_v, sem):
        wid = lax.axis_index("s") * _NC + lax.axis_index("c")
        a_base = wid * per_w
        tok_base = lax.rem(a_base, t)
        pltpu.sync_copy(pos_hbm.at[pl.ds(a_base, per_w)], pos_v)
        pltpu.sync_copy(x_hbm.at[pl.ds(tok_base, per_w)], rows_v)
        pltpu.async_copy(rows_v, xg_hbm.at[pos_v], sem).wait()

    return k(x, pos)


def _sc_combine(y, p1, p2, w1, w2):
    t = p1.shape[0]
    h = y.shape[1]
    per_w = t // _NW
    ncol = h // 16
    mesh = plsc.VectorSubcoreMesh(core_axis_name="c", subcore_axis_name="s")

    @functools.partial(
        pl.kernel,
        out_type=jax.ShapeDtypeStruct((t, h), jnp.float32),
        mesh=mesh,
        scratch_types=[
            pltpu.VMEM((per_w,), jnp.int32),
            pltpu.VMEM((per_w,), jnp.int32),
            pltpu.VMEM((per_w,), jnp.float32),
            pltpu.VMEM((per_w,), jnp.float32),
            pltpu.VMEM((per_w, h), jnp.float32),
            pltpu.VMEM((per_w, h), jnp.float32),
            pltpu.SemaphoreType.DMA,
        ],
        compiler_params=pltpu.CompilerParams(needs_layout_passes=False),
    )
    def k(y_hbm, p1_hbm, p2_hbm, w1_hbm, w2_hbm, out_hbm,
          i1_v, i2_v, a1_v, a2_v, r1_v, r2_v, sem):
        wid = lax.axis_index("s") * _NC + lax.axis_index("c")
        base = wid * per_w
        pltpu.sync_copy(p1_hbm.at[pl.ds(base, per_w)], i1_v)
        pltpu.sync_copy(p2_hbm.at[pl.ds(base, per_w)], i2_v)
        pltpu.sync_copy(w1_hbm.at[pl.ds(base, per_w)], a1_v)
        pltpu.sync_copy(w2_hbm.at[pl.ds(base, per_w)], a2_v)
        c1 = pltpu.async_copy(y_hbm.at[i1_v], r1_v, sem)
        c2 = pltpu.async_copy(y_hbm.at[i2_v], r2_v, sem)
        c1.wait()
        c2.wait()

        def body(tok, carry):
            splat = jnp.full((16,), tok, jnp.int32)
            s1 = plsc.load_gather(a1_v, [splat])
            s2 = plsc.load_gather(a2_v, [splat])
            for c in range(ncol):
                sl = pl.ds(c * 16, 16)
                r1_v[tok, sl] = s1 * r1_v[tok, sl] + s2 * r2_v[tok, sl]
            return carry

        lax.fori_loop(0, per_w, body, 0)
        pltpu.sync_copy(r1_v, out_hbm.at[pl.ds(base, per_w)])

    return k(y, p1, p2, w1, w2)


def _router_kernel(x_ref, gw_ref, logits_ref, meta_ref):
    x = x_ref[...]
    gw = gw_ref[...]
    l = jnp.dot(x, gw, preferred_element_type=jnp.float32)
    logits_ref[...] = l
    lane = jax.lax.broadcasted_iota(jnp.int32, l.shape, 1)
    valid = lane < 8
    neg = jnp.float32(-jnp.inf)
    lm = jnp.where(valid, l, neg)
    m1 = jnp.max(lm, axis=1, keepdims=True)
    i1 = jnp.min(jnp.where(lm == m1, lane, 128), axis=1, keepdims=True)
    s = jnp.sum(jnp.where(valid, jnp.exp(lm - m1), 0.0), axis=1, keepdims=True)
    lm2 = jnp.where(lane == i1, neg, lm)
    m2 = jnp.max(lm2, axis=1, keepdims=True)
    i2 = jnp.min(jnp.where(lm2 == m2, lane, 128), axis=1, keepdims=True)
    w1 = 1.0 / s
    w2 = jnp.exp(m2 - m1) / s
    meta = (jnp.where(lane == 0, i1.astype(jnp.float32), 0.0)
            + jnp.where(lane == 1, i2.astype(jnp.float32), 0.0)
            + jnp.where(lane == 2, w1, 0.0)
            + jnp.where(lane == 3, w2, 0.0))
    meta_ref[...] = meta


def _ffn_kernel(be_ref, nbu_ref, xg_ref, win_ref, wv_ref, wout_ref, y_ref):
    i = pl.program_id(0)
    bb = pl.program_id(1)
    ni = pl.num_programs(0)
    valid = bb < nbu_ref[0]

    @pl.when(valid)
    def _():
        xb = xg_ref[...]
        dotp = functools.partial(
            jax.lax.dot_general,
            dimension_numbers=(((1,), (0,)), ((), ())),
            precision=lax.Precision.DEFAULT,
            preferred_element_type=jnp.float32)
        up = dotp(xb, win_ref[0])
        v = dotp(xb, wv_ref[0])
        hg = jax.nn.gelu(up) * v
        part = dotp(hg, wout_ref[0])

        rows = pl.ds(bb * BLK, BLK)
        if ni == 1:
            y_ref[rows, :] = part
        else:
            @pl.when(i == 0)
            def _():
                y_ref[rows, :] = part

            @pl.when(i > 0)
            def _():
                y_ref[rows, :] += part


def kernel(hidden, gate_w, w_in, w_v, w_out):
    b, s, h = hidden.shape
    e = gate_w.shape[1]
    ii = w_in.shape[2]
    t = b * s
    x = hidden.reshape(t, h)

    gw_pad = jnp.zeros((h, 128), jnp.float32).at[:, :e].set(gate_w)
    rblk = 256
    logits_pad, meta = pl.pallas_call(
        _router_kernel,
        grid=(t // rblk,),
        in_specs=[
            pl.BlockSpec((rblk, h), lambda i: (i, 0)),
            pl.BlockSpec((h, 128), lambda i: (0, 0)),
        ],
        out_specs=[
            pl.BlockSpec((rblk, 128), lambda i: (i, 0)),
            pl.BlockSpec((rblk, 128), lambda i: (i, 0)),
        ],
        out_shape=[
            jax.ShapeDtypeStruct((t, 128), jnp.float32),
            jax.ShapeDtypeStruct((t, 128), jnp.float32),
        ],
    )(x, gw_pad)
    router_logits = logits_pad[:, :e]
    i1 = meta[:, 0].astype(jnp.int32)
    i2 = meta[:, 1].astype(jnp.int32)
    w1 = meta[:, 2]
    w2 = meta[:, 3]

    cap = 2 * t + e * BLK
    nb = cap // BLK
    ids = jnp.concatenate([i1, i2])
    onehot = (ids[:, None] == jnp.arange(e, dtype=jnp.int32)[None, :])
    csum = jnp.cumsum(onehot.astype(jnp.int32), axis=0)
    rank = jnp.sum(csum * onehot, axis=1) - 1
    counts = csum[-1]
    padded = ((counts + BLK - 1) // BLK) * BLK
    offs = jnp.concatenate([jnp.zeros((1,), jnp.int32),
                            jnp.cumsum(padded)[:-1].astype(jnp.int32)])
    pos = offs[ids] + rank
    blk_offs = offs // BLK
    block_expert = jnp.sum(
        jnp.arange(nb, dtype=jnp.int32)[:, None] >= blk_offs[None, :],
        axis=1).astype(jnp.int32) - 1
    block_expert = jnp.clip(block_expert, 0, e - 1)
    nb_used = jnp.sum(padded, dtype=jnp.int32) // BLK
    nb_used = nb_used.reshape(1)
    p1, p2 = pos[:t], pos[t:]

    xg = _sc_dispatch(x, pos, cap)

    ni = ii // TI
    grid_spec = pltpu.PrefetchScalarGridSpec(
        num_scalar_prefetch=2,
        grid=(ni, nb),
        in_specs=[
            pl.BlockSpec((BLK, h), lambda i, bb, be, nbu: (bb, 0)),
            pl.BlockSpec((1, h, TI), lambda i, bb, be, nbu: (be[bb], 0, i)),
            pl.BlockSpec((1, h, TI), lambda i, bb, be, nbu: (be[bb], 0, i)),
            pl.BlockSpec((1, TI, h), lambda i, bb, be, nbu: (be[bb], i, 0)),
        ],
        out_specs=pl.BlockSpec((cap, h), lambda i, bb, be, nbu: (0, 0)),
    )
    y = pl.pallas_call(
        _ffn_kernel,
        grid_spec=grid_spec,
        out_shape=jax.ShapeDtypeStruct((cap, h), jnp.float32),
        compiler_params=pltpu.CompilerParams(
            dimension_semantics=("arbitrary", "arbitrary"),
            vmem_limit_bytes=120 * 1024 * 1024),
    )(block_expert, nb_used, xg, w_in, w_v, w_out)

    out = _sc_combine(y, p1, p2, w1, w2)
    return out.reshape(b, s, h), router_logits


# speedup vs baseline: 1.1430x; 1.0077x over previous
import functools
import jax
import jax.numpy as jnp
from jax import lax
from jax.experimental import pallas as pl
from jax.experimental.pallas import tpu as pltpu
from jax.experimental.pallas import tpu_sc as plsc

BLK = 128
TI = 2048
_NC, _NS = 2, 16
_NW = _NC * _NS


def _sc_dispatch(x, pos, cap):
    t, h = x.shape
    na = 2 * t
    per_w = na // _NW
    mesh = plsc.VectorSubcoreMesh(core_axis_name="c", subcore_axis_name="s")

    @functools.partial(
        pl.kernel,
        out_type=jax.ShapeDtypeStruct((cap, h), jnp.float32),
        mesh=mesh,
        scratch_types=[
            pltpu.VMEM((per_w,), jnp.int32),
            pltpu.VMEM((per_w, h), jnp.float32),
            pltpu.SemaphoreType.DMA,
        ],
    )
    def k(x_hbm, pos_hbm, xg_hbm, pos_v, rows_v, sem):
        wid = lax.axis_index("s") * _NC + lax.axis_index("c")
        a_base = wid * per_w
        tok_base = lax.rem(a_base, t)
        pltpu.sync_copy(pos_hbm.at[pl.ds(a_base, per_w)], pos_v)
        pltpu.sync_copy(x_hbm.at[pl.ds(tok_base, per_w)], rows_v)
        pltpu.async_copy(rows_v, xg_hbm.at[pos_v], sem).wait()

    return k(x, pos)


def _sc_combine(y, p1, p2, w1, w2):
    t = p1.shape[0]
    h = y.shape[1]
    per_w = t // _NW
    ncol = h // 16
    mesh = plsc.VectorSubcoreMesh(core_axis_name="c", subcore_axis_name="s")

    @functools.partial(
        pl.kernel,
        out_type=jax.ShapeDtypeStruct((t, h), jnp.float32),
        mesh=mesh,
        scratch_types=[
            pltpu.VMEM((per_w,), jnp.int32),
            pltpu.VMEM((per_w,), jnp.int32),
            pltpu.VMEM((per_w,), jnp.float32),
            pltpu.VMEM((per_w,), jnp.float32),
            pltpu.VMEM((per_w, h), jnp.float32),
            pltpu.VMEM((per_w, h), jnp.float32),
            pltpu.SemaphoreType.DMA,
        ],
        compiler_params=pltpu.CompilerParams(needs_layout_passes=False),
    )
    def k(y_hbm, p1_hbm, p2_hbm, w1_hbm, w2_hbm, out_hbm,
          i1_v, i2_v, a1_v, a2_v, r1_v, r2_v, sem):
        wid = lax.axis_index("s") * _NC + lax.axis_index("c")
        base = wid * per_w
        pltpu.sync_copy(p1_hbm.at[pl.ds(base, per_w)], i1_v)
        pltpu.sync_copy(p2_hbm.at[pl.ds(base, per_w)], i2_v)
        pltpu.sync_copy(w1_hbm.at[pl.ds(base, per_w)], a1_v)
        pltpu.sync_copy(w2_hbm.at[pl.ds(base, per_w)], a2_v)
        c1 = pltpu.async_copy(y_hbm.at[i1_v], r1_v, sem)
        c2 = pltpu.async_copy(y_hbm.at[i2_v], r2_v, sem)
        c1.wait()
        c2.wait()

        def body(tok, carry):
            splat = jnp.full((16,), tok, jnp.int32)
            s1 = plsc.load_gather(a1_v, [splat])
            s2 = plsc.load_gather(a2_v, [splat])
            for c in range(ncol):
                sl = pl.ds(c * 16, 16)
                r1_v[tok, sl] = s1 * r1_v[tok, sl] + s2 * r2_v[tok, sl]
            return carry

        lax.fori_loop(0, per_w, body, 0)
        pltpu.sync_copy(r1_v, out_hbm.at[pl.ds(base, per_w)])

    return k(y, p1, p2, w1, w2)


def _router_kernel(x_ref, gw_ref, logits_ref, meta_ref):
    x = x_ref[...]
    gw = gw_ref[...]
    l = jnp.dot(x, gw, preferred_element_type=jnp.float32)
    logits_ref[...] = l
    lane = jax.lax.broadcasted_iota(jnp.int32, l.shape, 1)
    valid = lane < 8
    neg = jnp.float32(-jnp.inf)
    lm = jnp.where(valid, l, neg)
    m1 = jnp.max(lm, axis=1, keepdims=True)
    i1 = jnp.min(jnp.where(lm == m1, lane, 128), axis=1, keepdims=True)
    s = jnp.sum(jnp.where(valid, jnp.exp(lm - m1), 0.0), axis=1, keepdims=True)
    lm2 = jnp.where(lane == i1, neg, lm)
    m2 = jnp.max(lm2, axis=1, keepdims=True)
    i2 = jnp.min(jnp.where(lm2 == m2, lane, 128), axis=1, keepdims=True)
    w1 = 1.0 / s
    w2 = jnp.exp(m2 - m1) / s
    meta = (jnp.where(lane == 0, i1.astype(jnp.float32), 0.0)
            + jnp.where(lane == 1, i2.astype(jnp.float32), 0.0)
            + jnp.where(lane == 2, w1, 0.0)
            + jnp.where(lane == 3, w2, 0.0))
    meta_ref[...] = meta


def _meta_kernel(meta_ref, pos_ref, aux_ref):
    meta = meta_ref[...]
    tt = meta.shape[0]
    lane = lax.broadcasted_iota(jnp.int32, meta.shape, 1)
    lanef = lane.astype(jnp.float32)
    i1 = jnp.sum(jnp.where(lane == 0, meta, 0.0), axis=1, keepdims=True)
    i2 = jnp.sum(jnp.where(lane == 1, meta, 0.0), axis=1, keepdims=True)
    oh = (jnp.where(lanef == i1, 1.0, 0.0)
          + jnp.where(lanef == i2 + 8.0, 1.0, 0.0))
    rr = lax.broadcasted_iota(jnp.int32, (tt, tt), 0)
    cc = lax.broadcasted_iota(jnp.int32, (tt, tt), 1)
    tril = jnp.where(rr >= cc, 1.0, 0.0)
    csum = jnp.dot(tril, oh, preferred_element_type=jnp.float32)
    last = csum[tt - 1:tt, :]
    lr = lax.broadcasted_iota(jnp.int32, (128, 128), 0)
    lc = lax.broadcasted_iota(jnp.int32, (128, 128), 1)
    fold = jnp.where((lr == lc) | (lr == lc + 8), 1.0, 0.0)
    fold = fold * jnp.where(lc < 8, 1.0, 0.0)
    counts = jnp.dot(last, fold, preferred_element_type=jnp.float32)
    padded = jnp.ceil(counts / 128.0) * 128.0
    strict = jnp.where(lr < lc, 1.0, 0.0)
    offs = jnp.dot(padded, strict, preferred_element_type=jnp.float32)
    rank1 = jnp.sum(jnp.where(lanef == i1, csum, 0.0), axis=1,
                    keepdims=True) - 1.0
    cnt1_at_i2 = jnp.sum(jnp.where(lanef == i2, last, 0.0), axis=1,
                         keepdims=True)
    rank2 = cnt1_at_i2 + jnp.sum(
        jnp.where(lanef == i2 + 8.0, csum, 0.0), axis=1, keepdims=True) - 1.0
    off1 = jnp.sum(jnp.where(lanef == i1, offs, 0.0), axis=1, keepdims=True)
    off2 = jnp.sum(jnp.where(lanef == i2, offs, 0.0), axis=1, keepdims=True)
    pos1 = off1 + rank1
    pos2 = off2 + rank2
    pos = (jnp.where(lane == 0, pos1, 0.0)
           + jnp.where(lane == 1, pos2, 0.0))
    pos_ref[...] = pos.astype(jnp.int32)
    aux = (jnp.where(lax.broadcasted_iota(jnp.int32, (8, 128), 0) == 0,
                     counts, 0.0)
           + jnp.where(lax.broadcasted_iota(jnp.int32, (8, 128), 0) == 1,
                       padded, 0.0)
           + jnp.where(lax.broadcasted_iota(jnp.int32, (8, 128), 0) == 2,
                       offs, 0.0))
    aux_ref[...] = aux


def _ffn_kernel(be_ref, nbu_ref, xg_ref, win_ref, wv_ref, wout_ref, y_ref):
    i = pl.program_id(0)
    bb = pl.program_id(1)
    ni = pl.num_programs(0)
    valid = bb < nbu_ref[0]

    @pl.when(valid)
    def _():
        xb = xg_ref[...]
        dotp = functools.partial(
            jax.lax.dot_general,
            dimension_numbers=(((1,), (0,)), ((), ())),
            precision=lax.Precision.DEFAULT,
            preferred_element_type=jnp.float32)
        up = dotp(xb, win_ref[0])
        v = dotp(xb, wv_ref[0])
        hg = jax.nn.gelu(up) * v
        part = dotp(hg, wout_ref[0])

        rows = pl.ds(bb * BLK, BLK)
        if ni == 1:
            y_ref[rows, :] = part
        else:
            @pl.when(i == 0)
            def _():
                y_ref[rows, :] = part

            @pl.when(i > 0)
            def _():
                y_ref[rows, :] += part


def kernel(hidden, gate_w, w_in, w_v, w_out):
    b, s, h = hidden.shape
    e = gate_w.shape[1]
    ii = w_in.shape[2]
    t = b * s
    x = hidden.reshape(t, h)

    gw_pad = jnp.zeros((h, 128), jnp.float32).at[:, :e].set(gate_w)
    rblk = 256
    logits_pad, meta = pl.pallas_call(
        _router_kernel,
        grid=(t // rblk,),
        in_specs=[
            pl.BlockSpec((rblk, h), lambda i: (i, 0)),
            pl.BlockSpec((h, 128), lambda i: (0, 0)),
        ],
        out_specs=[
            pl.BlockSpec((rblk, 128), lambda i: (i, 0)),
            pl.BlockSpec((rblk, 128), lambda i: (i, 0)),
        ],
        out_shape=[
            jax.ShapeDtypeStruct((t, 128), jnp.float32),
            jax.ShapeDtypeStruct((t, 128), jnp.float32),
        ],
    )(x, gw_pad)
    router_logits = logits_pad[:, :e]
    w1 = meta[:, 2]
    w2 = meta[:, 3]

    cap = 2 * t + e * BLK
    nb = cap // BLK
    pos2d, aux = pl.pallas_call(
        _meta_kernel,
        out_shape=[
            jax.ShapeDtypeStruct((t, 128), jnp.int32),
            jax.ShapeDtypeStruct((8, 128), jnp.float32),
        ],
    )(meta)
    p1, p2 = pos2d[:, 0], pos2d[:, 1]
    pos = jnp.concatenate([p1, p2])
    blk_offs = (aux[2, :e] / BLK).astype(jnp.int32)
    block_expert = jnp.sum(
        jnp.arange(nb, dtype=jnp.int32)[:, None] >= blk_offs[None, :],
        axis=1).astype(jnp.int32) - 1
    block_expert = jnp.clip(block_expert, 0, e - 1)
    nb_used = ((aux[2, e - 1] + aux[1, e - 1]) / BLK).astype(jnp.int32)
    nb_used = nb_used.reshape(1)

    xg = _sc_dispatch(x, pos, cap)

    ni = ii // TI
    grid_spec = pltpu.PrefetchScalarGridSpec(
        num_scalar_prefetch=2,
        grid=(ni, nb),
        in_specs=[
            pl.BlockSpec((BLK, h), lambda i, bb, be, nbu: (bb, 0)),
            pl.BlockSpec((1, h, TI), lambda i, bb, be, nbu: (be[bb], 0, i)),
            pl.BlockSpec((1, h, TI), lambda i, bb, be, nbu: (be[bb], 0, i)),
            pl.BlockSpec((1, TI, h), lambda i, bb, be, nbu: (be[bb], i, 0)),
        ],
        out_specs=pl.BlockSpec((cap, h), lambda i, bb, be, nbu: (0, 0)),
    )
    y = pl.pallas_call(
        _ffn_kernel,
        grid_spec=grid_spec,
        out_shape=jax.ShapeDtypeStruct((cap, h), jnp.float32),
        compiler_params=pltpu.CompilerParams(
            dimension_semantics=("arbitrary", "arbitrary"),
            vmem_limit_bytes=120 * 1024 * 1024),
    )(block_expert, nb_used, xg, w_in, w_v, w_out)

    out = _sc_combine(y, p1, p2, w1, w2)
    return out.reshape(b, s, h), router_logits
